# Initial kernel scaffold; baseline (speedup 1.0000x reference)
import jax
import jax.numpy as jnp
from jax.experimental import pallas as pl


def kernel(feature, A, W1, b1, W2, b2, temp):
    raise NotImplementedError("write your pallas kernel here")


# speedup vs baseline: 3.0090x; 3.0090x over previous
import numpy as np
from math import comb

import jax
import jax.numpy as jnp
from jax.experimental import pallas as pl
from jax.experimental.pallas import tpu as pltpu

_K = 10
_N = 4096
_BLK = 128
_NBLK = _N // _BLK
_CH = 512
_NCH = _N // _CH

_COMB = np.array([comb(_K, j) / 2.0**_K for j in range(_K + 1)],
                 dtype=np.float32)


def _prep1_kernel(feat_ref, A_ref, W1t_ref, b1_ref, W2t_ref, b2_ref,
                  x_ref, dinv_ref):
    i = pl.program_id(0)
    h = jnp.dot(feat_ref[...].astype(jnp.bfloat16),
                W1t_ref[...].astype(jnp.bfloat16),
                preferred_element_type=jnp.float32)
    h = jnp.maximum(h + b1_ref[...], 0.0)
    x = jnp.dot(h.astype(jnp.bfloat16),
                W2t_ref[...].astype(jnp.bfloat16),
                preferred_element_type=jnp.float32) + b2_ref[...]
    x_ref[...] = x
    blk = A_ref[...]
    col = jax.lax.broadcasted_iota(jnp.int32, blk.shape, 1)
    row = jax.lax.broadcasted_iota(jnp.int32, blk.shape, 0) + i * _BLK
    blk0 = jnp.where(col == row, 0.0, blk)
    deg = jnp.sum(blk0, axis=1, keepdims=True)
    dinv_ref[...] = jnp.where(deg > 0.0, 1.0 / jnp.sqrt(deg), 0.0)


def _prep2_kernel(A_ref, dinvc_ref, dinvr_ref, An_ref):
    i = pl.program_id(0)
    blk = A_ref[...]
    col = jax.lax.broadcasted_iota(jnp.int32, blk.shape, 1)
    row = jax.lax.broadcasted_iota(jnp.int32, blk.shape, 0) + i * _BLK
    blk0 = jnp.where(col == row, 0.0, blk)
    An_ref[...] = ((dinvc_ref[...] * blk0) * dinvr_ref[...]).astype(jnp.bfloat16)


def _prop_kernel(coef_ref, An_hbm, x_ref, out_ref, An_vmem, B_scr, dma_sem):
    cp = pltpu.make_async_copy(An_hbm, An_vmem, dma_sem)
    cp.start()
    cp.wait()

    B_scr[:, 0:16] = x_ref[...]
    for j in range(1, _K + 1):
        c0 = 16 * (j - 1)

        def src(rows, j=j, c0=c0):
            if j == 1:
                return x_ref[rows, :]
            return B_scr[rows, c0:c0 + 16]

        tb = src(slice(None)).astype(jnp.bfloat16)

        def m2_body(c, carry, tb=tb, j=j, src=src):
            rows = pl.ds(c * _CH, _CH)
            T = jnp.dot(An_vmem[rows, :], tb, preferred_element_type=jnp.float32)
            piece = src(rows).astype(jnp.bfloat16).astype(jnp.float32) + T
            if j < _K:
                B_scr[rows, 16 * j:16 * j + 16] = piece
            else:
                out_ref[rows, :] = piece
            return carry

        jax.lax.fori_loop(0, _NCH, m2_body, 0)

    out = coef_ref[0] * out_ref[...]

    for k in range(1, _K + 1):
        W = 16 * (_K + 1 - k)
        Bb = B_scr[:, :W].astype(jnp.bfloat16)

        def l_body(c, carry, Bb=Bb, W=W):
            rows = pl.ds(c * _CH, _CH)
            T = jnp.dot(An_vmem[rows, :], Bb, preferred_element_type=jnp.float32)
            piece = B_scr[rows, :W].astype(jnp.bfloat16).astype(jnp.float32)
            B_scr[rows, :W] = piece - T
            return carry

        jax.lax.fori_loop(0, _NCH, l_body, 0)
        out = out + coef_ref[k] * B_scr[:, W - 16:W]

    out_ref[...] = out


def kernel(feature, A, W1, b1, W2, b2, temp):
    feature = feature.astype(jnp.float32)
    A = A.astype(jnp.float32)
    coef = jnp.asarray(_COMB) * jnp.maximum(temp.astype(jnp.float32), 0.0)

    x, dinv = pl.pallas_call(
        _prep1_kernel,
        grid=(_NBLK,),
        in_specs=[
            pl.BlockSpec((_BLK, 512), lambda i: (i, 0)),
            pl.BlockSpec((_BLK, _N), lambda i: (i, 0)),
            pl.BlockSpec((512, 256), lambda i: (0, 0)),
            pl.BlockSpec((1, 256), lambda i: (0, 0)),
            pl.BlockSpec((256, 16), lambda i: (0, 0)),
            pl.BlockSpec((1, 16), lambda i: (0, 0)),
        ],
        out_specs=[
            pl.BlockSpec((_BLK, 16), lambda i: (i, 0)),
            pl.BlockSpec((_BLK, 1), lambda i: (i, 0)),
        ],
        out_shape=[
            jax.ShapeDtypeStruct((_N, 16), jnp.float32),
            jax.ShapeDtypeStruct((_N, 1), jnp.float32),
        ],
    )(feature, A, W1.T, b1[None, :], W2.T, b2[None, :])

    An_b = pl.pallas_call(
        _prep2_kernel,
        grid=(_NBLK,),
        in_specs=[
            pl.BlockSpec((_BLK, _N), lambda i: (i, 0)),
            pl.BlockSpec((_BLK, 1), lambda i: (i, 0)),
            pl.BlockSpec((1, _N), lambda i: (0, 0)),
        ],
        out_specs=pl.BlockSpec((_BLK, _N), lambda i: (i, 0)),
        out_shape=jax.ShapeDtypeStruct((_N, _N), jnp.bfloat16),
    )(A, dinv, dinv.reshape(1, _N))

    out = pl.pallas_call(
        _prop_kernel,
        in_specs=[
            pl.BlockSpec(memory_space=pltpu.SMEM),
            pl.BlockSpec(memory_space=pl.ANY),
            pl.BlockSpec(memory_space=pltpu.VMEM),
        ],
        out_specs=pl.BlockSpec(memory_space=pltpu.VMEM),
        out_shape=jax.ShapeDtypeStruct((_N, 16), jnp.float32),
        scratch_shapes=[
            pltpu.VMEM((_N, _N), jnp.bfloat16),
            pltpu.VMEM((_N, 16 * _K), jnp.float32),
            pltpu.SemaphoreType.DMA,
        ],
    )(coef, An_b, x)
    return out


# speedup vs baseline: 3.6131x; 1.2008x over previous
import numpy as np
from math import comb

import jax
import jax.numpy as jnp
from jax.experimental import pallas as pl
from jax.experimental.pallas import tpu as pltpu

_K = 10
_N = 4096
_BLK = 128
_NBLK = _N // _BLK
_CH = 512
_NCH = _N // _CH

_COMB = np.array([comb(_K, j) / 2.0**_K for j in range(_K + 1)],
                 dtype=np.float32)


def _build_schedule(T=_K + 1, cap=8):
    rem = {p: _K - p for p in range(_K)}
    sched = []
    for t in range(1, T + 1):
        if t <= _K:
            cols = [t - 1]
            rem[t - 1] -= 1
        else:
            cols = []
        cand = [p for p in range(_K) if rem[p] > 0 and p + 1 < t]
        cand.sort(key=lambda p: (T - t) - rem[p])
        for p in cand[:cap - len(cols)]:
            cols.append(p)
            rem[p] -= 1
        sched.append(cols)
    assert all(v == 0 for v in rem.values()), rem
    return sched


_SCHED = _build_schedule()


def _prep1_kernel(feat_ref, A_ref, W1t_ref, b1_ref, W2t_ref, b2_ref,
                  x_ref, dinv_ref):
    i = pl.program_id(0)
    h = jnp.dot(feat_ref[...].astype(jnp.bfloat16),
                W1t_ref[...].astype(jnp.bfloat16),
                preferred_element_type=jnp.float32)
    h = jnp.maximum(h + b1_ref[...], 0.0)
    x = jnp.dot(h.astype(jnp.bfloat16),
                W2t_ref[...].astype(jnp.bfloat16),
                preferred_element_type=jnp.float32) + b2_ref[...]
    x_ref[...] = x
    blk = A_ref[...]
    sub = A_ref[:, pl.ds(i * _BLK, _BLK)]
    r = jax.lax.broadcasted_iota(jnp.int32, (_BLK, _BLK), 0)
    c = jax.lax.broadcasted_iota(jnp.int32, (_BLK, _BLK), 1)
    dg = jnp.sum(jnp.where(r == c, sub, 0.0), axis=1, keepdims=True)
    deg = jnp.sum(blk, axis=1, keepdims=True) - dg
    dinv_ref[...] = jnp.where(deg > 0.0, 1.0 / jnp.sqrt(deg), 0.0)


def _prep2_kernel(A_ref, dinvc_ref, dinvr_ref, An_ref):
    i = pl.program_id(0)
    An_ref[...] = ((dinvc_ref[...] * A_ref[...]) * dinvr_ref[...]).astype(jnp.bfloat16)
    r = jax.lax.broadcasted_iota(jnp.int32, (_BLK, _BLK), 0)
    c = jax.lax.broadcasted_iota(jnp.int32, (_BLK, _BLK), 1)
    sub = An_ref[:, pl.ds(i * _BLK, _BLK)]
    An_ref[:, pl.ds(i * _BLK, _BLK)] = jnp.where(r == c, jnp.bfloat16(0), sub)


def _prop_kernel(coef_ref, An_hbm, x_ref, out_ref, An_vmem, B_scr, Ob_scr,
                 dma_sem):
    cp = pltpu.make_async_copy(An_hbm, An_vmem, dma_sem)
    cp.start()
    cp.wait()

    B_scr[:, 0:16] = x_ref[...]
    out = None
    rem = {p: _K - p for p in range(_K)}
    for t in range(1, len(_SCHED) + 1):
        cols = _SCHED[t - 1]
        W = 16 * len(cols)
        for j, p in enumerate(cols):
            Ob_scr[:, 16 * j:16 * j + 16] = \
                B_scr[:, 16 * p:16 * p + 16].astype(jnp.bfloat16)
        Ob = Ob_scr[:, :W]

        def body(c, carry, Ob=Ob, cols=cols, t=t):
            rows = pl.ds(c * _CH, _CH)
            P = jnp.dot(An_vmem[rows, :], Ob, preferred_element_type=jnp.float32)
            for j, p in enumerate(cols):
                src = Ob_scr[rows, 16 * j:16 * j + 16].astype(jnp.float32)
                Pj = P[:, 16 * j:16 * j + 16]
                if t <= _K and p == t - 1:
                    if t < _K:
                        B_scr[rows, 16 * t:16 * t + 16] = src + Pj
                    else:
                        out_ref[rows, :] = src + Pj
                B_scr[rows, 16 * p:16 * p + 16] = src - Pj
            return carry

        jax.lax.fori_loop(0, _NCH, body, 0)

        for p in cols:
            rem[p] -= 1
            if rem[p] == 0:
                term = coef_ref[_K - p] * B_scr[:, 16 * p:16 * p + 16]
                out = term if out is None else out + term

    out = out + coef_ref[0] * out_ref[...]
    out_ref[...] = out


def kernel(feature, A, W1, b1, W2, b2, temp):
    feature = feature.astype(jnp.float32)
    A = A.astype(jnp.float32)
    coef = jnp.asarray(_COMB) * jnp.maximum(temp.astype(jnp.float32), 0.0)

    x, dinv = pl.pallas_call(
        _prep1_kernel,
        grid=(_NBLK,),
        in_specs=[
            pl.BlockSpec((_BLK, 512), lambda i: (i, 0)),
            pl.BlockSpec((_BLK, _N), lambda i: (i, 0)),
            pl.BlockSpec((512, 256), lambda i: (0, 0)),
            pl.BlockSpec((1, 256), lambda i: (0, 0)),
            pl.BlockSpec((256, 16), lambda i: (0, 0)),
            pl.BlockSpec((1, 16), lambda i: (0, 0)),
        ],
        out_specs=[
            pl.BlockSpec((_BLK, 16), lambda i: (i, 0)),
            pl.BlockSpec((_BLK, 1), lambda i: (i, 0)),
        ],
        out_shape=[
            jax.ShapeDtypeStruct((_N, 16), jnp.float32),
            jax.ShapeDtypeStruct((_N, 1), jnp.float32),
        ],
    )(feature, A, W1.T, b1[None, :], W2.T, b2[None, :])

    An_b = pl.pallas_call(
        _prep2_kernel,
        grid=(_NBLK,),
        in_specs=[
            pl.BlockSpec((_BLK, _N), lambda i: (i, 0)),
            pl.BlockSpec((_BLK, 1), lambda i: (i, 0)),
            pl.BlockSpec((1, _N), lambda i: (0, 0)),
        ],
        out_specs=pl.BlockSpec((_BLK, _N), lambda i: (i, 0)),
        out_shape=jax.ShapeDtypeStruct((_N, _N), jnp.bfloat16),
    )(A, dinv, dinv.reshape(1, _N))

    out = pl.pallas_call(
        _prop_kernel,
        in_specs=[
            pl.BlockSpec(memory_space=pltpu.SMEM),
            pl.BlockSpec(memory_space=pl.ANY),
            pl.BlockSpec(memory_space=pltpu.VMEM),
        ],
        out_specs=pl.BlockSpec(memory_space=pltpu.VMEM),
        out_shape=jax.ShapeDtypeStruct((_N, 16), jnp.float32),
        scratch_shapes=[
            pltpu.VMEM((_N, _N), jnp.bfloat16),
            pltpu.VMEM((_N, 16 * _K), jnp.float32),
            pltpu.VMEM((_N, 128), jnp.bfloat16),
            pltpu.SemaphoreType.DMA,
        ],
    )(coef, An_b, x)
    return out
